# baseline (device time: 44813 ns/iter reference)
import jax
import jax.numpy as jnp
from jax import lax
from jax.experimental import pallas as pl
from jax.experimental.pallas import tpu as pltpu

N_DEV = 32
ROWS = 64
RPS = ROWS // N_DEV


def kernel(x, Win0, Wout0, Win1, Wout1, Win2, Wout2):
    d_model = x.shape[1]
    d_ff = Win0.shape[1]

    def body(x_ref, win0_ref, wout0_ref, win1_ref, wout1_ref, win2_ref,
             wout2_ref, out_ref, psend, pbuf, agsend, xbuf, win_st, wout_st,
             send_sems, rs_sems, ag_sems, wsems):
        my = lax.axis_index("i")

        def stream_weights(win_ref, wout_ref, slot):
            cin = pltpu.make_async_copy(win_ref, win_st.at[slot],
                                        wsems.at[2 * slot])
            cout = pltpu.make_async_copy(wout_ref, wout_st.at[slot],
                                         wsems.at[2 * slot + 1])
            cin.start()
            cout.start()
            return cin, cout

        def partial_for(xv, slot, copies):
            for c in copies:
                c.wait()
            h = jnp.dot(xv, win_st[slot].astype(jnp.bfloat16),
                        preferred_element_type=jnp.float32)
            h = jnp.maximum(h, 0.0).astype(jnp.bfloat16)
            return jnp.dot(h, wout_st[slot].astype(jnp.bfloat16),
                           preferred_element_type=jnp.float32)

        def reduce_scatter(partial):
            psend[...] = partial.astype(jnp.bfloat16)
            sends = []
            for s in range(N_DEV):
                r = pltpu.make_async_remote_copy(
                    src_ref=psend.at[pl.ds(RPS * s, RPS)],
                    dst_ref=pbuf.at[my],
                    send_sem=send_sems.at[s],
                    recv_sem=rs_sems.at[my],
                    device_id=(s,),
                    device_id_type=pl.DeviceIdType.MESH,
                )
                r.start()
                sends.append(r)
            for p in range(N_DEV):
                pltpu.make_async_remote_copy(
                    src_ref=psend.at[pl.ds(0, RPS)],
                    dst_ref=pbuf.at[p],
                    send_sem=send_sems.at[p],
                    recv_sem=rs_sems.at[p],
                    device_id=(0,),
                    device_id_type=pl.DeviceIdType.MESH,
                ).wait_recv()
            red = jnp.sum(pbuf[...].astype(jnp.float32), axis=0)
            for r in sends:
                r.wait_send()
            return red

        def all_gather(red):
            agsend[...] = red.astype(jnp.bfloat16)
            sends = []
            for s in range(N_DEV):
                r = pltpu.make_async_remote_copy(
                    src_ref=agsend,
                    dst_ref=xbuf.at[pl.ds(RPS * my, RPS)],
                    send_sem=send_sems.at[s],
                    recv_sem=ag_sems.at[my],
                    device_id=(s,),
                    device_id_type=pl.DeviceIdType.MESH,
                )
                r.start()
                sends.append(r)
            for p in range(N_DEV):
                pltpu.make_async_remote_copy(
                    src_ref=agsend,
                    dst_ref=xbuf.at[pl.ds(RPS * p, RPS)],
                    send_sem=send_sems.at[p],
                    recv_sem=ag_sems.at[p],
                    device_id=(0,),
                    device_id_type=pl.DeviceIdType.MESH,
                ).wait_recv()
            xv = xbuf[...]
            for r in sends:
                r.wait_send()
            return xv

        w0 = stream_weights(win0_ref, wout0_ref, 0)
        barrier_sem = pltpu.get_barrier_semaphore()
        for d in range(1, N_DEV):
            pl.semaphore_signal(
                barrier_sem, inc=1,
                device_id=(lax.rem(my + d, N_DEV),),
                device_id_type=pl.DeviceIdType.MESH,
            )

        xv = x_ref[...].astype(jnp.bfloat16)
        partial = partial_for(xv, 0, w0)
        w1 = stream_weights(win1_ref, wout1_ref, 1)
        w2 = stream_weights(win2_ref, wout2_ref, 0)

        pl.semaphore_wait(barrier_sem, N_DEV - 1)

        red = reduce_scatter(partial)
        xv = all_gather(red)
        red = reduce_scatter(partial_for(xv, 1, w1))
        xv = all_gather(red)
        red = reduce_scatter(partial_for(xv, 0, w2))
        out_ref[...] = red

    return pl.pallas_call(
        body,
        out_shape=jax.ShapeDtypeStruct((RPS, d_model), jnp.float32),
        in_specs=[pl.BlockSpec(memory_space=pltpu.VMEM)]
        + [pl.BlockSpec(memory_space=pl.ANY)] * 6,
        out_specs=pl.BlockSpec(memory_space=pltpu.VMEM),
        scratch_shapes=[
            pltpu.VMEM((ROWS, d_model), jnp.bfloat16),
            pltpu.VMEM((N_DEV, RPS, d_model), jnp.bfloat16),
            pltpu.VMEM((RPS, d_model), jnp.bfloat16),
            pltpu.VMEM((ROWS, d_model), jnp.bfloat16),
            pltpu.VMEM((2, d_model, d_ff), jnp.float32),
            pltpu.VMEM((2, d_ff, d_model), jnp.float32),
            pltpu.SemaphoreType.DMA((N_DEV,)),
            pltpu.SemaphoreType.DMA((N_DEV,)),
            pltpu.SemaphoreType.DMA((N_DEV,)),
            pltpu.SemaphoreType.DMA((4,)),
        ],
        compiler_params=pltpu.CompilerParams(
            collective_id=0,
            vmem_limit_bytes=110 * 1024 * 1024,
        ),
    )(x, Win0, Wout0, Win1, Wout1, Win2, Wout2)


# device time: 44286 ns/iter; 1.0119x vs baseline; 1.0119x over previous
import jax
import jax.numpy as jnp
from jax import lax
from jax.experimental import pallas as pl
from jax.experimental.pallas import tpu as pltpu

N_DEV = 32
ROWS = 64
RPS = ROWS // N_DEV


def kernel(x, Win0, Wout0, Win1, Wout1, Win2, Wout2):
    d_model = x.shape[1]
    d_ff = Win0.shape[1]

    def body(x_ref, win0_ref, wout0_ref, win1_ref, wout1_ref, win2_ref,
             wout2_ref, out_ref, psend, pbuf, agsend, xbuf, win_st, wout_st,
             send_sems, rs_sems, ag_sems, wsems):
        my = lax.axis_index("i")

        def stream_weights(win_ref, wout_ref, slot):
            cin = pltpu.make_async_copy(win_ref, win_st.at[slot],
                                        wsems.at[2 * slot])
            cout = pltpu.make_async_copy(wout_ref, wout_st.at[slot],
                                         wsems.at[2 * slot + 1])
            cin.start()
            cout.start()
            return cin, cout

        def convert_weights(slot, copies):
            for c in copies:
                c.wait()
            return (win_st[slot].astype(jnp.bfloat16),
                    wout_st[slot].astype(jnp.bfloat16))

        def partial_for(xv, wbf):
            win_bf, wout_bf = wbf
            h = jnp.dot(xv, win_bf, preferred_element_type=jnp.float32)
            h = jnp.maximum(h, 0.0).astype(jnp.bfloat16)
            return jnp.dot(h, wout_bf, preferred_element_type=jnp.float32)

        def reduce_scatter(partial, filler=None):
            psend[...] = partial.astype(jnp.bfloat16)
            sends = []
            for s in range(N_DEV):
                r = pltpu.make_async_remote_copy(
                    src_ref=psend.at[pl.ds(RPS * s, RPS)],
                    dst_ref=pbuf.at[my],
                    send_sem=send_sems.at[s],
                    recv_sem=rs_sems.at[my],
                    device_id=(s,),
                    device_id_type=pl.DeviceIdType.MESH,
                )
                r.start()
                sends.append(r)
            aux = filler() if filler is not None else None
            for p in range(N_DEV):
                pltpu.make_async_remote_copy(
                    src_ref=psend.at[pl.ds(0, RPS)],
                    dst_ref=pbuf.at[p],
                    send_sem=send_sems.at[p],
                    recv_sem=rs_sems.at[p],
                    device_id=(0,),
                    device_id_type=pl.DeviceIdType.MESH,
                ).wait_recv()
            red = jnp.sum(pbuf[...].astype(jnp.float32), axis=0)
            for r in sends:
                r.wait_send()
            return red, aux

        def all_gather(red):
            agsend[...] = red.astype(jnp.bfloat16)
            sends = []
            for s in range(N_DEV):
                r = pltpu.make_async_remote_copy(
                    src_ref=agsend,
                    dst_ref=xbuf.at[pl.ds(RPS * my, RPS)],
                    send_sem=send_sems.at[s],
                    recv_sem=ag_sems.at[my],
                    device_id=(s,),
                    device_id_type=pl.DeviceIdType.MESH,
                )
                r.start()
                sends.append(r)
            for p in range(N_DEV):
                pltpu.make_async_remote_copy(
                    src_ref=agsend,
                    dst_ref=xbuf.at[pl.ds(RPS * p, RPS)],
                    send_sem=send_sems.at[p],
                    recv_sem=ag_sems.at[p],
                    device_id=(0,),
                    device_id_type=pl.DeviceIdType.MESH,
                ).wait_recv()
            xv = xbuf[...]
            for r in sends:
                r.wait_send()
            return xv

        w0_in, w0_out = stream_weights(win0_ref, wout0_ref, 0)
        barrier_sem = pltpu.get_barrier_semaphore()
        for d in range(1, N_DEV):
            pl.semaphore_signal(
                barrier_sem, inc=1,
                device_id=(lax.rem(my + d, N_DEV),),
                device_id_type=pl.DeviceIdType.MESH,
            )

        xv = x_ref[...].astype(jnp.bfloat16)
        w0_in.wait()
        h = jnp.dot(xv, win_st[0].astype(jnp.bfloat16),
                    preferred_element_type=jnp.float32)
        h = jnp.maximum(h, 0.0).astype(jnp.bfloat16)
        w0_out.wait()
        partial = jnp.dot(h, wout_st[0].astype(jnp.bfloat16),
                          preferred_element_type=jnp.float32)
        w1 = stream_weights(win1_ref, wout1_ref, 1)
        w2 = stream_weights(win2_ref, wout2_ref, 0)

        pl.semaphore_wait(barrier_sem, N_DEV - 1)

        red, w1bf = reduce_scatter(
            partial, filler=lambda: convert_weights(1, w1))
        xv = all_gather(red)
        red, w2bf = reduce_scatter(
            partial_for(xv, w1bf), filler=lambda: convert_weights(0, w2))
        xv = all_gather(red)
        red, _ = reduce_scatter(partial_for(xv, w2bf))
        out_ref[...] = red

    return pl.pallas_call(
        body,
        out_shape=jax.ShapeDtypeStruct((RPS, d_model), jnp.float32),
        in_specs=[pl.BlockSpec(memory_space=pltpu.VMEM)]
        + [pl.BlockSpec(memory_space=pl.ANY)] * 6,
        out_specs=pl.BlockSpec(memory_space=pltpu.VMEM),
        scratch_shapes=[
            pltpu.VMEM((ROWS, d_model), jnp.bfloat16),
            pltpu.VMEM((N_DEV, RPS, d_model), jnp.bfloat16),
            pltpu.VMEM((RPS, d_model), jnp.bfloat16),
            pltpu.VMEM((ROWS, d_model), jnp.bfloat16),
            pltpu.VMEM((2, d_model, d_ff), jnp.float32),
            pltpu.VMEM((2, d_ff, d_model), jnp.float32),
            pltpu.SemaphoreType.DMA((N_DEV,)),
            pltpu.SemaphoreType.DMA((N_DEV,)),
            pltpu.SemaphoreType.DMA((N_DEV,)),
            pltpu.SemaphoreType.DMA((4,)),
        ],
        compiler_params=pltpu.CompilerParams(
            collective_id=0,
            vmem_limit_bytes=110 * 1024 * 1024,
        ),
    )(x, Win0, Wout0, Win1, Wout1, Win2, Wout2)
